# baseline (speedup 1.0000x reference)
import functools

import jax
import jax.numpy as jnp
from jax import lax
from jax.experimental import pallas as pl
from jax.experimental.pallas import tpu as pltpu
from jax.experimental.pallas import tpu_sc as plsc

N = 10000
E = 320000
D = 128

NC = 2
NS = 16
NW = NC * NS
C = 128
NCH = 80
H0 = 40
H1 = NCH - H0
EP = NCH * C
E_PAD = NW * EP
N_PAD = 10240
ROWS_PT = N_PAD // NS
ZCP = ROWS_PT // C

_mesh = plsc.VectorSubcoreMesh(core_axis_name="c", subcore_axis_name="s")


@functools.partial(
    pl.kernel,
    out_type=jax.ShapeDtypeStruct((NC, 2, N_PAD), jnp.float32),
    mesh=_mesh,
    scratch_types=[
        pltpu.VMEM((NCH, C), jnp.int32),
        pltpu.VMEM((NCH, C), jnp.int32),
        pltpu.VMEM((C,), jnp.float32),
        pltpu.VMEM_SHARED((N_PAD,), jnp.float32),
        pltpu.VMEM_SHARED((N_PAD,), jnp.float32),
    ],
)
def _deg_kernel(edge_hbm, ones_hbm, zeros_hbm, out_hbm,
                sidx, didx, ones_v, acc_out, acc_in):
    cid = lax.axis_index("c")
    sid = lax.axis_index("s")
    wid = cid * NS + sid
    zslice = pl.ds(sid * (N_PAD // NS), N_PAD // NS)
    pltpu.sync_copy(zeros_hbm.at[zslice], acc_out.at[zslice])
    pltpu.sync_copy(zeros_hbm.at[zslice], acc_in.at[zslice])
    pltpu.sync_copy(edge_hbm.at[0, wid], sidx)
    pltpu.sync_copy(edge_hbm.at[1, wid], didx)
    pltpu.sync_copy(ones_hbm, ones_v)
    plsc.subcore_barrier()

    def body(j, carry):
        pltpu.sync_copy(ones_v, acc_out.at[sidx.at[j]], add=True)
        pltpu.sync_copy(ones_v, acc_in.at[didx.at[j]], add=True)
        return carry

    lax.fori_loop(0, NCH, body, 0)
    plsc.subcore_barrier()
    pltpu.sync_copy(acc_out.at[zslice], out_hbm.at[cid, 0, zslice])
    pltpu.sync_copy(acc_in.at[zslice], out_hbm.at[cid, 1, zslice])


@functools.partial(
    pl.kernel,
    out_type=jax.ShapeDtypeStruct((NC, N_PAD, D), jnp.float32),
    mesh=_mesh,
    scratch_types=[
        pltpu.VMEM((H0, C), jnp.int32),
        pltpu.VMEM((H0, C), jnp.int32),
        pltpu.VMEM((C, D), jnp.float32),
        pltpu.VMEM((C, D), jnp.float32),
        pltpu.VMEM_SHARED((N_PAD, D), jnp.float32),
        pltpu.SemaphoreType.DMA,
        pltpu.SemaphoreType.DMA,
    ],
)
def _conv_kernel(t_hbm, edge_hbm, zeros_hbm, out_hbm,
                 sidx, didx, rows0, rows1, acc, gsem0, gsem1):
    cid = lax.axis_index("c")
    sid = lax.axis_index("s")
    wid = cid * NS + sid
    zslice = pl.ds(sid * ROWS_PT, ROWS_PT)
    pltpu.sync_copy(zeros_hbm, rows0)
    for k in range(ZCP):
        pltpu.async_copy(
            rows0, acc.at[pl.ds(sid * ROWS_PT + k * C, C)], gsem0)
    pltpu.async_copy(edge_hbm.at[0, wid, pl.ds(0, H0)], sidx, gsem1)
    pltpu.async_copy(edge_hbm.at[1, wid, pl.ds(0, H0)], didx, gsem1)
    for k in range(ZCP):
        pltpu.make_async_copy(
            rows0, acc.at[pl.ds(sid * ROWS_PT + k * C, C)], gsem0).wait()
    pltpu.make_async_copy(edge_hbm.at[0, wid, pl.ds(0, H0)], sidx,
                          gsem1).wait()
    pltpu.make_async_copy(edge_hbm.at[1, wid, pl.ds(0, H0)], didx,
                          gsem1).wait()
    plsc.subcore_barrier()

    def make_body(npairs):
        def body(jj, carry):
            j0 = 2 * jj
            pltpu.async_copy(t_hbm.at[sidx.at[j0 + 1]], rows1, gsem1)
            pltpu.make_async_copy(t_hbm.at[sidx.at[j0]], rows0, gsem0).wait()
            pltpu.sync_copy(rows0, acc.at[didx.at[j0]], add=True)

            @pl.when(jj < npairs - 1)
            def _():
                pltpu.async_copy(t_hbm.at[sidx.at[j0 + 2]], rows0, gsem0)

            pltpu.make_async_copy(t_hbm.at[sidx.at[j0 + 1]], rows1,
                                  gsem1).wait()
            pltpu.sync_copy(rows1, acc.at[didx.at[j0 + 1]], add=True)
            return carry
        return body

    pltpu.async_copy(t_hbm.at[sidx.at[0]], rows0, gsem0)
    lax.fori_loop(0, H0 // 2, make_body(H0 // 2), 0)

    pltpu.sync_copy(edge_hbm.at[0, wid, pl.ds(H0, H1)], sidx)
    pltpu.sync_copy(edge_hbm.at[1, wid, pl.ds(H0, H1)], didx)
    pltpu.async_copy(t_hbm.at[sidx.at[0]], rows0, gsem0)
    lax.fori_loop(0, H1 // 2, make_body(H1 // 2), 0)

    plsc.subcore_barrier()
    pltpu.sync_copy(acc.at[zslice], out_hbm.at[cid, zslice])


BR = 2000


def _tc_mm_body(h_ref, w0_ref, q_ref):
    q_ref[...] = jnp.dot(h_ref[...], w0_ref[...],
                         preferred_element_type=jnp.float32)


BC = 2048
PAD_PER_ROW = (E_PAD - E) // 240


def _tc1_body(d00_ref, d10_ref, d01_ref, d11_ref, q_ref,
              csrc_ref, cdst_ref, t0_ref):
    i = pl.program_id(0)
    row = lax.broadcasted_iota(jnp.int32, (BC, 1), 0) + i * BC
    corr = jnp.where(row < 240, float(PAD_PER_ROW), 0.0)
    deg_out = d00_ref[...] + d10_ref[...] - corr
    deg_in = d01_ref[...] + d11_ref[...]
    c_src = lax.rsqrt(jnp.maximum(deg_out, 1.0))
    c_dst = lax.rsqrt(jnp.maximum(deg_in, 1.0))
    csrc_ref[...] = c_src
    cdst_ref[...] = c_dst
    t0_ref[...] = q_ref[...] * c_src


def _tc_mid_body(p_ref, cdst_ref, b_ref, csrc_ref, w_ref, t_ref):
    agg = (p_ref[0] + p_ref[1]) * cdst_ref[...]
    hnew = jnp.maximum(agg + b_ref[...], 0.0)
    t_ref[...] = jnp.dot(hnew * csrc_ref[...], w_ref[...],
                         preferred_element_type=jnp.float32)


def _tc_post_body(p_ref, cdst_ref, b_ref, out_ref):
    agg = (p_ref[0] + p_ref[1]) * cdst_ref[...]
    out_ref[...] = jnp.maximum(agg + b_ref[...], 0.0)


_rows = pl.BlockSpec((BR, D), lambda i: (i, 0))
_col = pl.BlockSpec((BR, 1), lambda i: (i, 0))
_wfull = pl.BlockSpec((D, D), lambda i: (0, 0))
_bfull = pl.BlockSpec((1, D), lambda i: (0, 0))
_pblk = pl.BlockSpec((NC, BR, D), lambda i: (0, i, 0))
_SEC = N_PAD // BC


def _degsec(k):
    return pl.BlockSpec((BC, 1), lambda i: (i + _SEC * k, 0))

_tc_mm = pl.pallas_call(
    _tc_mm_body,
    grid=(N // BR,),
    in_specs=[_rows, _wfull],
    out_specs=_rows,
    out_shape=jax.ShapeDtypeStruct((N, D), jnp.float32),
)

_tc1 = pl.pallas_call(
    _tc1_body,
    grid=(_SEC,),
    in_specs=[_degsec(0), _degsec(2), _degsec(1), _degsec(3),
              pl.BlockSpec((BC, D), lambda i: (i, 0))],
    out_specs=(pl.BlockSpec((BC, 1), lambda i: (i, 0)),
               pl.BlockSpec((BC, 1), lambda i: (i, 0)),
               pl.BlockSpec((BC, D), lambda i: (i, 0))),
    out_shape=(
        jax.ShapeDtypeStruct((N, 1), jnp.float32),
        jax.ShapeDtypeStruct((N, 1), jnp.float32),
        jax.ShapeDtypeStruct((N, D), jnp.float32),
    ),
)

_tc_mid = pl.pallas_call(
    _tc_mid_body,
    grid=(N // BR,),
    in_specs=[_pblk, _col, _bfull, _col, _wfull],
    out_specs=_rows,
    out_shape=jax.ShapeDtypeStruct((N, D), jnp.float32),
)

_tc_post = pl.pallas_call(
    _tc_post_body,
    grid=(N // BR,),
    in_specs=[_pblk, _col, _bfull],
    out_specs=_rows,
    out_shape=jax.ShapeDtypeStruct((N, D), jnp.float32),
)


def kernel(h, edge_index, W0, b0, W1, b1):
    pad = jnp.arange(E_PAD - E, dtype=jnp.int32) % 240
    pad = jnp.stack([pad, pad + N])
    edges_raw = edge_index.astype(jnp.int32)
    edges = jnp.concatenate([edges_raw, pad], axis=1)
    edges = edges.reshape(2, NW, NCH, C)
    ones_c = jnp.ones((C,), jnp.float32)
    zeros_pad = jnp.zeros((N_PAD,), jnp.float32)
    zeros_cd = jnp.zeros((C, D), jnp.float32)

    q0 = _tc_mm(h, W0)
    deg = _deg_kernel(edges, ones_c, zeros_pad)
    deg2 = deg.reshape(NC * 2 * N_PAD, 1)
    c_src, c_dst, t0 = _tc1(deg2, deg2, deg2, deg2, q0)

    p0 = _conv_kernel(t0, edges, zeros_cd)
    t1 = _tc_mid(p0, c_dst, b0.reshape(1, D), c_src, W1)

    p1 = _conv_kernel(t1, edges, zeros_cd)
    return _tc_post(p1, c_dst, b1.reshape(1, D))


# speedup vs baseline: 1.0631x; 1.0631x over previous
import functools

import jax
import jax.numpy as jnp
from jax import lax
from jax.experimental import pallas as pl
from jax.experimental.pallas import tpu as pltpu
from jax.experimental.pallas import tpu_sc as plsc

N = 10000
E = 320000
D = 128

NC = 2
NS = 16
NW = NC * NS
C = 128
NCH = 80
H0 = 40
H1 = NCH - H0
EP = NCH * C
E_PAD = NW * EP
N_PAD = 10240
ROWS_PT = N_PAD // NS
ZCP = ROWS_PT // C

_mesh = plsc.VectorSubcoreMesh(core_axis_name="c", subcore_axis_name="s")


@functools.partial(
    pl.kernel,
    out_type=jax.ShapeDtypeStruct((NC, 2, N_PAD), jnp.float32),
    mesh=_mesh,
    scratch_types=[
        pltpu.VMEM((NCH, C), jnp.int32),
        pltpu.VMEM((NCH, C), jnp.int32),
        pltpu.VMEM((C,), jnp.float32),
        pltpu.VMEM_SHARED((N_PAD,), jnp.float32),
        pltpu.VMEM_SHARED((N_PAD,), jnp.float32),
    ],
)
def _deg_kernel(edge_hbm, ones_hbm, zeros_hbm, out_hbm,
                sidx, didx, ones_v, acc_out, acc_in):
    cid = lax.axis_index("c")
    sid = lax.axis_index("s")
    wid = cid * NS + sid
    zslice = pl.ds(sid * (N_PAD // NS), N_PAD // NS)
    pltpu.sync_copy(zeros_hbm.at[zslice], acc_out.at[zslice])
    pltpu.sync_copy(zeros_hbm.at[zslice], acc_in.at[zslice])
    pltpu.sync_copy(edge_hbm.at[0, wid], sidx)
    pltpu.sync_copy(edge_hbm.at[1, wid], didx)
    pltpu.sync_copy(ones_hbm, ones_v)
    plsc.subcore_barrier()

    def body(j, carry):
        pltpu.sync_copy(ones_v, acc_out.at[sidx.at[j]], add=True)
        pltpu.sync_copy(ones_v, acc_in.at[didx.at[j]], add=True)
        return carry

    lax.fori_loop(0, NCH, body, 0)
    plsc.subcore_barrier()
    pltpu.sync_copy(acc_out.at[zslice], out_hbm.at[cid, 0, zslice])
    pltpu.sync_copy(acc_in.at[zslice], out_hbm.at[cid, 1, zslice])


@functools.partial(
    pl.kernel,
    out_type=jax.ShapeDtypeStruct((NC, N_PAD, D), jnp.float32),
    mesh=_mesh,
    scratch_types=[
        pltpu.VMEM((H0, C), jnp.int32),
        pltpu.VMEM((H0, C), jnp.int32),
        pltpu.VMEM((C, D), jnp.float32),
        pltpu.VMEM((C, D), jnp.float32),
        pltpu.VMEM_SHARED((N_PAD, D), jnp.float32),
        pltpu.SemaphoreType.DMA,
        pltpu.SemaphoreType.DMA,
    ],
)
def _conv_kernel(t_hbm, edge_hbm, zeros_hbm, out_hbm,
                 sidx, didx, rows0, rows1, acc, gsem0, gsem1):
    cid = lax.axis_index("c")
    sid = lax.axis_index("s")
    wid = cid * NS + sid
    zslice = pl.ds(sid * ROWS_PT, ROWS_PT)
    pltpu.sync_copy(zeros_hbm, rows0)
    for k in range(ZCP):
        pltpu.async_copy(
            rows0, acc.at[pl.ds(sid * ROWS_PT + k * C, C)], gsem0)
    pltpu.async_copy(edge_hbm.at[0, wid, pl.ds(0, H0)], sidx, gsem1)
    pltpu.async_copy(edge_hbm.at[1, wid, pl.ds(0, H0)], didx, gsem1)
    for k in range(ZCP):
        pltpu.make_async_copy(
            rows0, acc.at[pl.ds(sid * ROWS_PT + k * C, C)], gsem0).wait()
    pltpu.make_async_copy(edge_hbm.at[0, wid, pl.ds(0, H0)], sidx,
                          gsem1).wait()
    pltpu.make_async_copy(edge_hbm.at[1, wid, pl.ds(0, H0)], didx,
                          gsem1).wait()
    plsc.subcore_barrier()

    def make_body(npairs):
        def body(jj, carry):
            j0 = 2 * jj
            pltpu.async_copy(t_hbm.at[sidx.at[j0 + 1]], rows1, gsem1)
            pltpu.make_async_copy(t_hbm.at[sidx.at[j0]], rows0, gsem0).wait()
            pltpu.sync_copy(rows0, acc.at[didx.at[j0]], add=True)

            @pl.when(jj < npairs - 1)
            def _():
                pltpu.async_copy(t_hbm.at[sidx.at[j0 + 2]], rows0, gsem0)

            pltpu.make_async_copy(t_hbm.at[sidx.at[j0 + 1]], rows1,
                                  gsem1).wait()
            pltpu.sync_copy(rows1, acc.at[didx.at[j0 + 1]], add=True)
            return carry
        return body

    pltpu.async_copy(t_hbm.at[sidx.at[0]], rows0, gsem0)
    lax.fori_loop(0, H0 // 2, make_body(H0 // 2), 0)

    pltpu.sync_copy(edge_hbm.at[0, wid, pl.ds(H0, H1)], sidx)
    pltpu.sync_copy(edge_hbm.at[1, wid, pl.ds(H0, H1)], didx)
    pltpu.async_copy(t_hbm.at[sidx.at[0]], rows0, gsem0)
    lax.fori_loop(0, H1 // 2, make_body(H1 // 2), 0)

    plsc.subcore_barrier()
    pltpu.sync_copy(acc.at[zslice], out_hbm.at[cid, zslice])


BR = 2000


def _tc_mm_body(h_ref, w0_ref, q_ref):
    q_ref[...] = jnp.dot(h_ref[...], w0_ref[...],
                         preferred_element_type=jnp.float32)


PAD_PER_ROW = (E_PAD - E) // 240


def _tc1_body(deg_ref, q_ref, csrc_ref, cdst_ref, t0_ref):
    i = pl.program_id(0)
    row = lax.broadcasted_iota(jnp.int32, (BR, 1), 0) + i * BR
    corr = jnp.where(row < 240, float(PAD_PER_ROW), 0.0)
    deg_out = deg_ref[0, 0] + deg_ref[1, 0] - corr
    deg_in = deg_ref[0, 1] + deg_ref[1, 1]
    c_src = lax.rsqrt(jnp.maximum(deg_out, 1.0))
    c_dst = lax.rsqrt(jnp.maximum(deg_in, 1.0))
    csrc_ref[...] = c_src
    cdst_ref[...] = c_dst
    t0_ref[...] = q_ref[...] * c_src


def _tc_mid_body(p_ref, cdst_ref, b_ref, csrc_ref, w_ref, t_ref):
    agg = (p_ref[0] + p_ref[1]) * cdst_ref[...]
    hnew = jnp.maximum(agg + b_ref[...], 0.0)
    t_ref[...] = jnp.dot(hnew * csrc_ref[...], w_ref[...],
                         preferred_element_type=jnp.float32)


def _tc_post_body(p_ref, cdst_ref, b_ref, out_ref):
    agg = (p_ref[0] + p_ref[1]) * cdst_ref[...]
    out_ref[...] = jnp.maximum(agg + b_ref[...], 0.0)


_rows = pl.BlockSpec((BR, D), lambda i: (i, 0))
_col = pl.BlockSpec((BR, 1), lambda i: (i, 0))
_wfull = pl.BlockSpec((D, D), lambda i: (0, 0))
_bfull = pl.BlockSpec((1, D), lambda i: (0, 0))
_pblk = pl.BlockSpec((NC, BR, D), lambda i: (0, i, 0))
_degblk = pl.BlockSpec((NC, 2, BR, 1), lambda i: (0, 0, i, 0))

_tc_mm = pl.pallas_call(
    _tc_mm_body,
    grid=(N // BR,),
    in_specs=[_rows, _wfull],
    out_specs=_rows,
    out_shape=jax.ShapeDtypeStruct((N, D), jnp.float32),
)

_tc1 = pl.pallas_call(
    _tc1_body,
    grid=(N // BR,),
    in_specs=[_degblk, _rows],
    out_specs=(_col, _col, _rows),
    out_shape=(
        jax.ShapeDtypeStruct((N, 1), jnp.float32),
        jax.ShapeDtypeStruct((N, 1), jnp.float32),
        jax.ShapeDtypeStruct((N, D), jnp.float32),
    ),
)

_tc_mid = pl.pallas_call(
    _tc_mid_body,
    grid=(N // BR,),
    in_specs=[_pblk, _col, _bfull, _col, _wfull],
    out_specs=_rows,
    out_shape=jax.ShapeDtypeStruct((N, D), jnp.float32),
)

_tc_post = pl.pallas_call(
    _tc_post_body,
    grid=(N // BR,),
    in_specs=[_pblk, _col, _bfull],
    out_specs=_rows,
    out_shape=jax.ShapeDtypeStruct((N, D), jnp.float32),
)


def kernel(h, edge_index, W0, b0, W1, b1):
    pad = jnp.arange(E_PAD - E, dtype=jnp.int32) % 240
    pad = jnp.stack([pad, pad + N])
    edges_raw = edge_index.astype(jnp.int32)
    edges = jnp.concatenate([edges_raw, pad], axis=1)
    edges = edges.reshape(2, NW, NCH, C)
    ones_c = jnp.ones((C,), jnp.float32)
    zeros_pad = jnp.zeros((N_PAD,), jnp.float32)
    zeros_cd = jnp.zeros((C, D), jnp.float32)

    q0 = _tc_mm(h, W0)
    deg = _deg_kernel(edges, ones_c, zeros_pad)
    deg4 = deg.reshape(NC, 2, N_PAD, 1)
    c_src, c_dst, t0 = _tc1(deg4, q0)

    p0 = _conv_kernel(t0, edges, zeros_cd)
    t1 = _tc_mid(p0, c_dst, b0.reshape(1, D), c_src, W1)

    p1 = _conv_kernel(t1, edges, zeros_cd)
    return _tc_post(p1, c_dst, b1.reshape(1, D))
